# baseline (speedup 1.0000x reference)
import jax
import jax.numpy as jnp
from jax import lax
from jax.experimental import pallas as pl
from jax.experimental.pallas import tpu as pltpu
from jax.experimental.pallas import tpu_sc as plsc

_TL_UNKNOWN = 5
_CROSSWALK = 10
_LANE_BDRY_LEFT = 11
_LANE_BDRY_RIGHT = 12
_NUM_TYPES = 13
_D = 64

_B = 4096
_TOTAL = 194
_BDRY_START = 66
_HEAD = 8
_CONST = _TOTAL - _HEAD

_NC, _NS, _L = 2, 16, 16
_NW = _NC * _NS
_RPT = _B // _NW
_KB = 4
_NGRP = _RPT // _KB


def _body(tl_hbm, tpad_hbm, table_hbm, out_hbm,
          tl_v, idx_v, rows_v, const_v, head0_v, head1_v, table_v,
          gsem, bsem, hsem0, hsem1):
    wid = lax.axis_index("s") * _NC + lax.axis_index("c")
    b0 = wid * _RPT

    pltpu.sync_copy(tl_hbm.at[pl.ds(b0, _RPT)], tl_v)
    pltpu.sync_copy(table_hbm, table_v)

    for q in range(_RPT // _L):
        t16 = tl_v[pl.ds(q * _L, _L)]
        i16 = jnp.clip(t16.astype(jnp.int32) + _TL_UNKNOWN, 0, _NUM_TYPES - 1)
        idx_v[pl.ds(q * _L, _L)] = i16

    rows_cp = pltpu.async_copy(tpad_hbm.at[idx_v], rows_v, gsem)

    r10 = [table_v[_CROSSWALK, pl.ds(q * _L, _L)] for q in range(_D // _L)]
    r11 = [table_v[_LANE_BDRY_LEFT, pl.ds(q * _L, _L)] for q in range(_D // _L)]
    r12 = [table_v[_LANE_BDRY_RIGHT, pl.ds(q * _L, _L)] for q in range(_D // _L)]

    def fill(c, carry):
        is_cw = c < _BDRY_START
        is_right = ((c - _BDRY_START) & 1) == 1
        for q in range(_D // _L):
            v = jnp.where(is_cw, r10[q], jnp.where(is_right, r12[q], r11[q]))
            for k in range(_KB):
                const_v[k, c - _HEAD, pl.ds(q * _L, _L)] = v
        return carry

    lax.fori_loop(_HEAD, _TOTAL, fill, 0)

    for head in (head0_v, head1_v):
        for j in range(_KB):
            for c in range(2, _HEAD):
                for q in range(_D // _L):
                    head[j, c, pl.ds(q * _L, _L)] = r10[q]

    rows_cp.wait()

    heads = (head0_v, head1_v)
    hsems = (hsem0, hsem1)
    head_cps = [None, None]
    const_cps = []
    for g in range(_NGRP):
        bg = b0 + g * _KB
        const_cps.append(pltpu.async_copy(
            const_v, out_hbm.at[pl.ds(bg, _KB), pl.ds(_HEAD, _CONST), :],
            bsem))
        head = heads[g & 1]
        if head_cps[g & 1] is not None:
            head_cps[g & 1].wait()
        for j in range(_KB):
            r = g * _KB + j
            for q in range(_D // _L):
                v = rows_v[r, pl.ds(q * _L, _L)]
                head[j, 0, pl.ds(q * _L, _L)] = v
                head[j, 1, pl.ds(q * _L, _L)] = v
        head_cps[g & 1] = pltpu.async_copy(
            head, out_hbm.at[pl.ds(bg, _KB), pl.ds(0, _HEAD), :],
            hsems[g & 1])
    for cp in const_cps:
        cp.wait()
    head_cps[0].wait()
    head_cps[1].wait()


@jax.jit
def _emb_fill(tl, table_pad, table):
    fn = pl.kernel(
        _body,
        out_type=jax.ShapeDtypeStruct((_B, _TOTAL, _D), jnp.float32),
        mesh=plsc.VectorSubcoreMesh(core_axis_name="c", subcore_axis_name="s"),
        scratch_types=[
            pltpu.VMEM((_RPT,), jnp.float32),
            pltpu.VMEM((_RPT,), jnp.int32),
            pltpu.VMEM((_RPT, 2 * _D), jnp.float32),
            pltpu.VMEM((_KB, _CONST, _D), jnp.float32),
            pltpu.VMEM((_KB, _HEAD, _D), jnp.float32),
            pltpu.VMEM((_KB, _HEAD, _D), jnp.float32),
            pltpu.VMEM((_NUM_TYPES, _D), jnp.float32),
            pltpu.SemaphoreType.DMA,
            pltpu.SemaphoreType.DMA,
            pltpu.SemaphoreType.DMA,
            pltpu.SemaphoreType.DMA,
        ],
    )
    return fn(tl, table_pad, table)


def kernel(type, lanes_mid, crosswalks, lanes, emb_table):
    del type, crosswalks, lanes
    tl = lanes_mid[:, 0, 0, -1]
    table_pad = jnp.pad(emb_table, ((0, 0), (0, _D)))
    return _emb_fill(tl, table_pad, emb_table)


# speedup vs baseline: 6.8860x; 6.8860x over previous
import jax
import jax.numpy as jnp
from jax import lax
from jax.experimental import pallas as pl
from jax.experimental.pallas import tpu as pltpu
from jax.experimental.pallas import tpu_sc as plsc

_TL_UNKNOWN = 5
_CROSSWALK = 10
_NUM_TYPES = 13
_D = 64

_B = 4096
_TOTAL = 194
_BDRY_START = 66
_CPT = 6
_W = 256

_NC, _NS, _L = 2, 16, 16
_NW = _NC * _NS
_BPT = _B // _NW


def _take(v, i):
    return lax.gather(
        v, i[:, None],
        dimension_numbers=lax.GatherDimensionNumbers(
            offset_dims=(), collapsed_slice_dims=(0,), start_index_map=(0,)),
        slice_sizes=(1,),
        mode=lax.GatherScatterMode.PROMISE_IN_BOUNDS)


def _body(tl_hbm, tableT_hbm, trep_hbm, out_hbm,
          tl_v, tmpl_v, vbuf_v, tableT_v, trep_v, osem):
    wid = lax.axis_index("s") * _NC + lax.axis_index("c")
    b0 = wid * _BPT

    pltpu.sync_copy(tl_hbm.at[pl.ds(b0, _BPT)], tl_v)
    pltpu.sync_copy(tableT_hbm, tableT_v)
    pltpu.sync_copy(trep_hbm, trep_v)

    idx = []
    for k in range(_BPT // _L):
        t16 = tl_v[pl.ds(k * _L, _L)]
        idx.append(jnp.clip(t16.astype(jnp.int32) + _TL_UNKNOWN,
                            0, _NUM_TYPES - 1))

    for t in range(3):
        for d in range(_D):
            v = trep_v[t, d, pl.ds(0, _L)]
            for k in range(_W // _L):
                tmpl_v[t, d, pl.ds(k * _L, _L)] = v

    for d in range(_D):
        col_d = tableT_v[d, pl.ds(0, _L)]
        for k in range(_BPT // _L):
            vbuf_v[d, pl.ds(k * _L, _L)] = _take(col_d, idx[k])

    cps = [
        pltpu.async_copy(vbuf_v, out_hbm.at[0, :, pl.ds(b0, _BPT)], osem),
        pltpu.async_copy(vbuf_v, out_hbm.at[1, :, pl.ds(b0, _BPT)], osem),
    ]
    for j in range(_CPT):
        c = 2 + _CPT * wid + j
        rsel = jnp.where(c < _BDRY_START, 0, 1 + ((c - _BDRY_START) & 1))
        for k in range(_B // _W):
            cps.append(pltpu.async_copy(
                tmpl_v.at[rsel], out_hbm.at[c, :, pl.ds(k * _W, _W)], osem))
    for cp in cps:
        cp.wait()


@jax.jit
def _emb_fill(tl, tableT, trep):
    fn = pl.kernel(
        _body,
        out_type=jax.ShapeDtypeStruct((_TOTAL, _D, _B), jnp.float32),
        mesh=plsc.VectorSubcoreMesh(core_axis_name="c", subcore_axis_name="s"),
        scratch_types=[
            pltpu.VMEM((_BPT,), jnp.float32),
            pltpu.VMEM((3, _D, _W), jnp.float32),
            pltpu.VMEM((_D, _BPT), jnp.float32),
            pltpu.VMEM((_D, _L), jnp.float32),
            pltpu.VMEM((3, _D, _L), jnp.float32),
            pltpu.SemaphoreType.DMA,
        ],
    )
    outT = fn(tl, tableT, trep)
    return jnp.transpose(outT, (2, 0, 1))


def kernel(type, lanes_mid, crosswalks, lanes, emb_table):
    del type, crosswalks, lanes
    tl = lanes_mid[:, 0, 0, -1]
    tableT = jnp.pad(emb_table.T, ((0, 0), (0, _L - _NUM_TYPES)))
    trep = jnp.broadcast_to(
        emb_table[_CROSSWALK:_CROSSWALK + 3, :, None], (3, _D, _L))
    return _emb_fill(tl, tableT, trep)


# speedup vs baseline: 7.0551x; 1.0246x over previous
import jax
import jax.numpy as jnp
from jax import lax
from jax.experimental import pallas as pl
from jax.experimental.pallas import tpu as pltpu
from jax.experimental.pallas import tpu_sc as plsc

_TL_UNKNOWN = 5
_CROSSWALK = 10
_NUM_TYPES = 13
_D = 64

_B = 4096
_TOTAL = 194
_BDRY_START = 66
_CPT = 6
_W = 512

_NC, _NS, _L = 2, 16, 16
_NW = _NC * _NS
_BPT = _B // _NW


def _take(v, i):
    return lax.gather(
        v, i[:, None],
        dimension_numbers=lax.GatherDimensionNumbers(
            offset_dims=(), collapsed_slice_dims=(0,), start_index_map=(0,)),
        slice_sizes=(1,),
        mode=lax.GatherScatterMode.PROMISE_IN_BOUNDS)


def _body(tl_hbm, tableT_hbm, trep_hbm, out_hbm,
          tl_v, tmpl_v, vbuf_v, tableT_v, osem):
    wid = lax.axis_index("s") * _NC + lax.axis_index("c")
    b0 = wid * _BPT

    pltpu.sync_copy(tl_hbm.at[pl.ds(b0, _BPT)], tl_v)
    pltpu.sync_copy(tableT_hbm, tableT_v)
    pltpu.sync_copy(trep_hbm, tmpl_v)

    idx = []
    for k in range(_BPT // _L):
        t16 = tl_v[pl.ds(k * _L, _L)]
        idx.append(jnp.clip(t16.astype(jnp.int32) + _TL_UNKNOWN,
                            0, _NUM_TYPES - 1))

    for d in range(_D):
        col_d = tableT_v[d, pl.ds(0, _L)]
        for k in range(_BPT // _L):
            vbuf_v[d, pl.ds(k * _L, _L)] = _take(col_d, idx[k])

    cps = [
        pltpu.async_copy(vbuf_v, out_hbm.at[0, :, pl.ds(b0, _BPT)], osem),
        pltpu.async_copy(vbuf_v, out_hbm.at[1, :, pl.ds(b0, _BPT)], osem),
    ]
    for j in range(_CPT):
        c = 2 + _CPT * wid + j
        rsel = jnp.where(c < _BDRY_START, 0, 1 + ((c - _BDRY_START) & 1))
        for k in range(_B // _W):
            cps.append(pltpu.async_copy(
                tmpl_v.at[rsel], out_hbm.at[c, :, pl.ds(k * _W, _W)], osem))
    for cp in cps:
        cp.wait()


@jax.jit
def _emb_fill(tl, tableT, trep):
    fn = pl.kernel(
        _body,
        out_type=jax.ShapeDtypeStruct((_TOTAL, _D, _B), jnp.float32),
        mesh=plsc.VectorSubcoreMesh(core_axis_name="c", subcore_axis_name="s"),
        scratch_types=[
            pltpu.VMEM((_BPT,), jnp.float32),
            pltpu.VMEM((3, _D, _W), jnp.float32),
            pltpu.VMEM((_D, _BPT), jnp.float32),
            pltpu.VMEM((_D, _L), jnp.float32),
            pltpu.SemaphoreType.DMA,
        ],
    )
    outT = fn(tl, tableT, trep)
    return jnp.transpose(outT, (2, 0, 1))


def kernel(type, lanes_mid, crosswalks, lanes, emb_table):
    del type, crosswalks, lanes
    tl = lanes_mid[:, 0, 0, -1]
    tableT = jnp.pad(emb_table.T, ((0, 0), (0, _L - _NUM_TYPES)))
    trep = jnp.broadcast_to(
        emb_table[_CROSSWALK:_CROSSWALK + 3, :, None], (3, _D, _W))
    return _emb_fill(tl, tableT, trep)


# speedup vs baseline: 7.1029x; 1.0068x over previous
import jax
import jax.numpy as jnp
from jax import lax
from jax.experimental import pallas as pl
from jax.experimental.pallas import tpu as pltpu
from jax.experimental.pallas import tpu_sc as plsc

_TL_UNKNOWN = 5
_CROSSWALK = 10
_NUM_TYPES = 13
_D = 64

_B = 4096
_TOTAL = 194
_BDRY_START = 66
_CPT = 6
_W = 512

_NC, _NS, _L = 2, 16, 16
_NW = _NC * _NS
_BPT = _B // _NW


def _take(v, i):
    return lax.gather(
        v, i[:, None],
        dimension_numbers=lax.GatherDimensionNumbers(
            offset_dims=(), collapsed_slice_dims=(0,), start_index_map=(0,)),
        slice_sizes=(1,),
        mode=lax.GatherScatterMode.PROMISE_IN_BOUNDS)


def _body(tl_hbm, tableT_hbm, trep_hbm, out_hbm,
          tl_v, tmpl_v, vbuf_v, tableT_v, tsem, osem):
    wid = lax.axis_index("s") * _NC + lax.axis_index("c")
    b0 = wid * _BPT

    tcp = pltpu.async_copy(trep_hbm, tmpl_v, tsem)
    pltpu.sync_copy(tl_hbm.at[pl.ds(b0, _BPT)], tl_v)
    pltpu.sync_copy(tableT_hbm, tableT_v)

    idx = []
    for k in range(_BPT // _L):
        t16 = tl_v[pl.ds(k * _L, _L)]
        idx.append(jnp.clip(t16.astype(jnp.int32) + _TL_UNKNOWN,
                            0, _NUM_TYPES - 1))

    for d in range(_D):
        col_d = tableT_v[d, pl.ds(0, _L)]
        for k in range(_BPT // _L):
            vbuf_v[d, pl.ds(k * _L, _L)] = _take(col_d, idx[k])

    cps = [
        pltpu.async_copy(vbuf_v, out_hbm.at[0, :, pl.ds(b0, _BPT)], osem),
        pltpu.async_copy(vbuf_v, out_hbm.at[1, :, pl.ds(b0, _BPT)], osem),
    ]
    tcp.wait()
    for j in range(_CPT):
        c = 2 + _CPT * wid + j
        rsel = jnp.where(c < _BDRY_START, 0, 1 + ((c - _BDRY_START) & 1))
        for k in range(_B // _W):
            cps.append(pltpu.async_copy(
                tmpl_v.at[rsel], out_hbm.at[c, :, pl.ds(k * _W, _W)], osem))
    for cp in cps:
        cp.wait()


@jax.jit
def _emb_fill(tl, tableT, trep):
    fn = pl.kernel(
        _body,
        out_type=jax.ShapeDtypeStruct((_TOTAL, _D, _B), jnp.float32),
        mesh=plsc.VectorSubcoreMesh(core_axis_name="c", subcore_axis_name="s"),
        scratch_types=[
            pltpu.VMEM((_BPT,), jnp.float32),
            pltpu.VMEM((3, _D, _W), jnp.float32),
            pltpu.VMEM((_D, _BPT), jnp.float32),
            pltpu.VMEM((_D, _L), jnp.float32),
            pltpu.SemaphoreType.DMA,
            pltpu.SemaphoreType.DMA,
        ],
    )
    outT = fn(tl, tableT, trep)
    return jnp.transpose(outT, (2, 0, 1))


def kernel(type, lanes_mid, crosswalks, lanes, emb_table):
    del type, crosswalks, lanes
    tl = lanes_mid[:, 0, 0, -1]
    tableT = jnp.pad(emb_table.T, ((0, 0), (0, _L - _NUM_TYPES)))
    trep = jnp.broadcast_to(
        emb_table[_CROSSWALK:_CROSSWALK + 3, :, None], (3, _D, _W))
    return _emb_fill(tl, tableT, trep)


# speedup vs baseline: 7.4009x; 1.0419x over previous
import jax
import jax.numpy as jnp
from jax import lax
from jax.experimental import pallas as pl
from jax.experimental.pallas import tpu as pltpu
from jax.experimental.pallas import tpu_sc as plsc

_TL_UNKNOWN = 5
_CROSSWALK = 10
_NUM_TYPES = 13
_D = 64

_B = 4096
_TOTAL = 194
_BDRY_START = 66
_CPT = 6
_W = 512
_SWAP_A, _SWAP_B = 67, 68

_NC, _NS, _L = 2, 16, 16
_NW = _NC * _NS
_BPT = _B // _NW


def _take(v, i):
    return lax.gather(
        v, i[:, None],
        dimension_numbers=lax.GatherDimensionNumbers(
            offset_dims=(), collapsed_slice_dims=(0,), start_index_map=(0,)),
        slice_sizes=(1,),
        mode=lax.GatherScatterMode.PROMISE_IN_BOUNDS)


def _body(tl_hbm, tableT_hbm, trep_hbm, out_hbm,
          tl_v, tmpl_v, vbuf_v, tableT_v, tsem, osem):
    wid = lax.axis_index("s") * _NC + lax.axis_index("c")
    b0 = wid * _BPT

    base = 2 + _CPT * wid
    cols, rows = [], []
    for j in range(_CPT):
        bj = base + j
        c = jnp.where(bj == _SWAP_A, _SWAP_B,
                      jnp.where(bj == _SWAP_B, _SWAP_A, bj))
        cols.append(c)
        rows.append(jnp.where(c < _BDRY_START, 0, 1 + ((c - _BDRY_START) & 1)))
    row_lo, row_hi = rows[0], rows[0]
    for r in rows[1:]:
        row_lo = jnp.minimum(row_lo, r)
        row_hi = jnp.maximum(row_hi, r)

    tcp0 = pltpu.async_copy(trep_hbm.at[row_lo], tmpl_v.at[0], tsem)
    tcp1 = pltpu.async_copy(trep_hbm.at[row_hi], tmpl_v.at[1], tsem)
    pltpu.sync_copy(tl_hbm.at[pl.ds(b0, _BPT)], tl_v)
    pltpu.sync_copy(tableT_hbm, tableT_v)

    idx = []
    for k in range(_BPT // _L):
        t16 = tl_v[pl.ds(k * _L, _L)]
        idx.append(jnp.clip(t16.astype(jnp.int32) + _TL_UNKNOWN,
                            0, _NUM_TYPES - 1))

    for d in range(_D):
        col_d = tableT_v[d, pl.ds(0, _L)]
        for k in range(_BPT // _L):
            vbuf_v[d, pl.ds(k * _L, _L)] = _take(col_d, idx[k])

    cps = [
        pltpu.async_copy(vbuf_v, out_hbm.at[0, :, pl.ds(b0, _BPT)], osem),
        pltpu.async_copy(vbuf_v, out_hbm.at[1, :, pl.ds(b0, _BPT)], osem),
    ]
    tcp0.wait()
    tcp1.wait()
    for j in range(_CPT):
        rsel = jnp.where(rows[j] == row_hi, 1, 0)
        for k in range(_B // _W):
            cps.append(pltpu.async_copy(
                tmpl_v.at[rsel], out_hbm.at[cols[j], :, pl.ds(k * _W, _W)],
                osem))
    for cp in cps:
        cp.wait()


@jax.jit
def _emb_fill(tl, tableT, trep):
    fn = pl.kernel(
        _body,
        out_type=jax.ShapeDtypeStruct((_TOTAL, _D, _B), jnp.float32),
        mesh=plsc.VectorSubcoreMesh(core_axis_name="c", subcore_axis_name="s"),
        scratch_types=[
            pltpu.VMEM((_BPT,), jnp.float32),
            pltpu.VMEM((2, _D, _W), jnp.float32),
            pltpu.VMEM((_D, _BPT), jnp.float32),
            pltpu.VMEM((_D, _L), jnp.float32),
            pltpu.SemaphoreType.DMA,
            pltpu.SemaphoreType.DMA,
        ],
    )
    outT = fn(tl, tableT, trep)
    return jnp.transpose(outT, (2, 0, 1))


def kernel(type, lanes_mid, crosswalks, lanes, emb_table):
    del type, crosswalks, lanes
    tl = lanes_mid[:, 0, 0, -1]
    tableT = jnp.pad(emb_table.T, ((0, 0), (0, _L - _NUM_TYPES)))
    trep = jnp.broadcast_to(
        emb_table[_CROSSWALK:_CROSSWALK + 3, :, None], (3, _D, _W))
    return _emb_fill(tl, tableT, trep)


# speedup vs baseline: 7.5892x; 1.0254x over previous
import jax
import jax.numpy as jnp
from jax import lax
from jax.experimental import pallas as pl
from jax.experimental.pallas import tpu as pltpu
from jax.experimental.pallas import tpu_sc as plsc

_TL_UNKNOWN = 5
_CROSSWALK = 10
_NUM_TYPES = 13
_D = 64

_B = 4096
_TOTAL = 194
_BDRY_START = 66
_CPT = 6
_W = 512
_SWAP_A, _SWAP_B = 67, 68

_NC, _NS, _L = 2, 16, 16
_NW = _NC * _NS
_BPT = _B // _NW


def _take(v, i):
    return lax.gather(
        v, i[:, None],
        dimension_numbers=lax.GatherDimensionNumbers(
            offset_dims=(), collapsed_slice_dims=(0,), start_index_map=(0,)),
        slice_sizes=(1,),
        mode=lax.GatherScatterMode.PROMISE_IN_BOUNDS)


def _body(tl_hbm, pack_hbm, out_hbm, tl_v, tmpl_v, vbuf_v, tab2_v,
          tsem, osem):
    wid = lax.axis_index("s") * _NC + lax.axis_index("c")
    b0 = wid * _BPT

    base = 2 + _CPT * wid
    cols, rows = [], []
    for j in range(_CPT):
        bj = base + j
        c = jnp.where(bj == _SWAP_A, _SWAP_B,
                      jnp.where(bj == _SWAP_B, _SWAP_A, bj))
        cols.append(c)
        rows.append(jnp.where(c < _BDRY_START, 0, 1 + ((c - _BDRY_START) & 1)))
    row_lo, row_hi = rows[0], rows[0]
    for r in rows[1:]:
        row_lo = jnp.minimum(row_lo, r)
        row_hi = jnp.maximum(row_hi, r)

    tcp0 = pltpu.async_copy(
        pack_hbm.at[pl.ds(row_lo * _D, _D)], tmpl_v.at[0], tsem)
    tcp1 = pltpu.async_copy(
        pack_hbm.at[pl.ds(row_hi * _D, _D)], tmpl_v.at[1], tsem)
    pltpu.sync_copy(tl_hbm.at[pl.ds(b0, _BPT)], tl_v)
    pltpu.sync_copy(pack_hbm.at[pl.ds(3 * _D, 8)], tab2_v)

    idx = []
    for k in range(_BPT // _L):
        t16 = tl_v[pl.ds(k * _L, _L)]
        idx.append(jnp.clip(t16.astype(jnp.int32) + _TL_UNKNOWN,
                            0, _NUM_TYPES - 1))

    for d in range(_D):
        col_d = tab2_v[d // 32, pl.ds((d % 32) * _L, _L)]
        for k in range(_BPT // _L):
            vbuf_v[d, pl.ds(k * _L, _L)] = _take(col_d, idx[k])

    cps = [
        pltpu.async_copy(vbuf_v, out_hbm.at[0, :, pl.ds(b0, _BPT)], osem),
        pltpu.async_copy(vbuf_v, out_hbm.at[1, :, pl.ds(b0, _BPT)], osem),
    ]
    tcp0.wait()
    tcp1.wait()
    for j in range(_CPT):
        rsel = jnp.where(rows[j] == row_hi, 1, 0)
        for k in range(_B // _W):
            cps.append(pltpu.async_copy(
                tmpl_v.at[rsel], out_hbm.at[cols[j], :, pl.ds(k * _W, _W)],
                osem))
    for cp in cps:
        cp.wait()


@jax.jit
def _emb_fill(tl, pack):
    fn = pl.kernel(
        _body,
        out_type=jax.ShapeDtypeStruct((_TOTAL, _D, _B), jnp.float32),
        mesh=plsc.VectorSubcoreMesh(core_axis_name="c", subcore_axis_name="s"),
        scratch_types=[
            pltpu.VMEM((_BPT,), jnp.float32),
            pltpu.VMEM((2, _D, _W), jnp.float32),
            pltpu.VMEM((_D, _BPT), jnp.float32),
            pltpu.VMEM((8, _W), jnp.float32),
            pltpu.SemaphoreType.DMA,
            pltpu.SemaphoreType.DMA,
        ],
    )
    outT = fn(tl, pack)
    return jnp.transpose(outT, (2, 0, 1))


def kernel(type, lanes_mid, crosswalks, lanes, emb_table):
    del type, crosswalks, lanes
    tl = lanes_mid[:, 0, 0, -1]
    tabT = jnp.pad(emb_table.T, ((0, 0), (0, _L - _NUM_TYPES)))
    pack = jnp.concatenate([
        jnp.broadcast_to(
            emb_table[_CROSSWALK:_CROSSWALK + 3, :, None],
            (3, _D, _W)).reshape(3 * _D, _W),
        tabT.reshape(2, _W),
        jnp.zeros((6, _W), jnp.float32),
    ], axis=0)
    return _emb_fill(tl, pack)
